# baseline (speedup 1.0000x reference)
import functools

import jax
import jax.numpy as jnp
import numpy as np
from jax import lax
from jax.experimental import pallas as pl
from jax.experimental.pallas import tpu as pltpu
from jax.experimental.pallas import tpu_sc as plsc

N_ATOMS = 10000
N_EDGES = 320000
N_BASIS = 128
N_FILTERS = 128
N_INTER = 3
N_RBF = 20
CUTOFF = 5.0

_NC = 2
_NS = 16
_NW = _NC * _NS
_CH = 128
_NCHUNK = 79
_EPW = _CH * _NCHUNK
_EPAD = _EPW * _NW
_HALF0 = 6000
_TRASH0 = 6000
_TRASH1 = 4112
_ACC = 6400
_RPT = _ACC // _NS

_NP_A = 10240

_RBF_PAD = 24
_OFFS = np.linspace(0.0, CUTOFF, N_RBF)
_COEFF = -0.5 / (_OFFS[1] - _OFFS[0]) ** 2
_OFFS_P = np.zeros((1, _RBF_PAD), np.float32)
_OFFS_P[0, :N_RBF] = _OFFS
_LOG2 = float(np.log(2.0))

_BE = 2048
_NB = _EPAD // _BE
_BR = 2000


def _ssp(x):
    return jnp.maximum(x, 0.0) + jnp.log1p(jnp.exp(-jnp.abs(x))) - _LOG2


_PCH = 1264
_ZPT = _NP_A // _NW


def _prep_body(rx_hbm, ry_hbm, rz_hbm, idxi_hbm, idxj_hbm, z_hbm, emb_hbm,
               dsq_hbm, x0_hbm,
               rx_v, ry_v, rz_v, ii_v, jj_v, dsq_v, zz_v, x0_v, sem):
    cid = lax.axis_index("c")
    sid = lax.axis_index("s")
    wid = cid * _NS + sid
    e0 = wid * _EPW

    pltpu.sync_copy(rx_hbm, rx_v)
    pltpu.sync_copy(ry_hbm, ry_v)
    pltpu.sync_copy(rz_hbm, rz_v)

    pltpu.sync_copy(z_hbm.at[pl.ds(wid * _ZPT, _ZPT)], zz_v)
    for q in range(4):
        pltpu.async_copy(emb_hbm.at[zz_v.at[pl.ds(q * 80, 80)]], x0_v,
                         sem).wait()
        pltpu.sync_copy(x0_v, x0_hbm.at[pl.ds(wid * _ZPT + q * 80, 80)])

    zf = jnp.zeros((16,), jnp.float32)

    def zrow(i, carry):
        for r in range(1, 8):
            dsq_v[r, pl.ds(i * 16, 16)] = zf
        return carry

    lax.fori_loop(0, _EPW // 16, zrow, 0)

    for c in range(8):
        pltpu.sync_copy(idxi_hbm.at[pl.ds(e0 + c * _PCH, _PCH)], ii_v)
        pltpu.sync_copy(idxj_hbm.at[pl.ds(e0 + c * _PCH, _PCH)], jj_v)

        def grp(g, carry):
            sl = pl.ds(g * 16, 16)
            vi = ii_v[sl]
            vj = jj_v[sl]
            dx = plsc.load_gather(rx_v, [vi]) - plsc.load_gather(rx_v, [vj])
            dy = plsc.load_gather(ry_v, [vi]) - plsc.load_gather(ry_v, [vj])
            dz = plsc.load_gather(rz_v, [vi]) - plsc.load_gather(rz_v, [vj])
            dsq_v[0, pl.ds(c * _PCH + g * 16, 16)] = dx * dx + dy * dy + dz * dz
            return carry

        lax.fori_loop(0, _PCH // 16, grp, 0)

    pltpu.sync_copy(dsq_v, dsq_hbm.at[:, pl.ds(e0, _EPW)])


_prep_call = functools.partial(
    pl.kernel,
    out_type=(jax.ShapeDtypeStruct((8, _EPAD), jnp.float32),
              jax.ShapeDtypeStruct((_NP_A, N_BASIS), jnp.float32)),
    mesh=plsc.VectorSubcoreMesh(core_axis_name="c", subcore_axis_name="s"),
    compiler_params=pltpu.CompilerParams(needs_layout_passes=False),
    scratch_types=[
        pltpu.VMEM((N_ATOMS,), jnp.float32),
        pltpu.VMEM((N_ATOMS,), jnp.float32),
        pltpu.VMEM((N_ATOMS,), jnp.float32),
        pltpu.VMEM((_PCH,), jnp.int32),
        pltpu.VMEM((_PCH,), jnp.int32),
        pltpu.VMEM((8, _EPW), jnp.float32),
        pltpu.VMEM((_ZPT,), jnp.int32),
        pltpu.VMEM((80, N_BASIS), jnp.float32),
        pltpu.SemaphoreType.DMA,
    ],
)(_prep_body)


def _edge_body(h_hbm, wij_hbm, idxi_hbm, idxj_hbm, out_hbm,
               idxa_v, idxb_v, idxj_v, xj_v, wij_v, s_sh, sem):
    cid = lax.axis_index("c")
    sid = lax.axis_index("s")
    wid = cid * _NS + sid
    r0 = sid * _RPT

    pltpu.sync_copy(idxi_hbm.at[wid], idxa_v)
    pltpu.sync_copy(idxj_hbm.at[pl.ds(wid * _EPW, _EPW)], idxj_v)
    half = jnp.full((16,), _HALF0, jnp.int32)
    trash0 = jnp.full((16,), _TRASH0, jnp.int32)
    trash1 = jnp.full((16,), _TRASH1, jnp.int32)

    def remap_row(j, carry):
        for k in range(8):
            sl = pl.ds(k * 16, 16)
            v = idxa_v[j, sl]
            in_a = v < half
            idxb_v[j, sl] = jnp.where(in_a, trash1, v - half)
            idxa_v[j, sl] = jnp.where(in_a, v, trash0)
        return carry

    lax.fori_loop(0, _NCHUNK, remap_row, 0)

    zf = jnp.zeros((16,), jnp.float32)

    def zero_row(r, carry):
        for k in range(8):
            xj_v[r, pl.ds(k * 16, 16)] = zf
        return carry

    for p in range(2):
        lax.fori_loop(0, _CH, zero_row, 0)
        for j in range(3):
            pltpu.sync_copy(xj_v, s_sh.at[pl.ds(r0 + j * _CH, _CH)])
        pltpu.sync_copy(xj_v.at[pl.ds(0, 16)], s_sh.at[pl.ds(r0 + 384, 16)])
        plsc.subcore_barrier()

        idx_ref = idxa_v if p == 0 else idxb_v

        def chunk(j, carry):
            pltpu.async_copy(h_hbm.at[idxj_v.at[pl.ds(j * _CH, _CH)]], xj_v,
                             sem).wait()
            pltpu.sync_copy(wij_hbm.at[wid * _NCHUNK + j], wij_v)

            def mul_row(r, c2):
                for k in range(8):
                    sl = pl.ds(k * 16, 16)
                    xj_v[r, sl] = xj_v[r, sl] * wij_v[r, sl]
                return c2

            lax.fori_loop(0, _CH, mul_row, 0)
            pltpu.sync_copy(xj_v, s_sh.at[idx_ref.at[j]], add=True)
            return carry

        lax.fori_loop(0, _NCHUNK, chunk, 0)

        plsc.subcore_barrier()
        pltpu.sync_copy(s_sh.at[pl.ds(r0, _RPT)],
                        out_hbm.at[cid, p, pl.ds(r0, _RPT)])
        plsc.subcore_barrier()


_edge_call = functools.partial(
    pl.kernel,
    out_type=jax.ShapeDtypeStruct((_NC, 2, _ACC, N_FILTERS), jnp.float32),
    mesh=plsc.VectorSubcoreMesh(core_axis_name="c", subcore_axis_name="s"),
    scratch_types=[
        pltpu.VMEM((_NCHUNK, _CH), jnp.int32),
        pltpu.VMEM((_NCHUNK, _CH), jnp.int32),
        pltpu.VMEM((_EPW,), jnp.int32),
        pltpu.VMEM((_CH, N_FILTERS), jnp.float32),
        pltpu.VMEM((_CH, N_FILTERS), jnp.float32),
        pltpu.VMEM_SHARED((_ACC, N_FILTERS), jnp.float32),
        pltpu.SemaphoreType.DMA,
    ],
)(_edge_body)


def _filt_body(dsq_ref, w1_ref, b1_ref, w2_ref, b2_ref, out_ref):
    b = pl.program_id(1)
    dsq8 = dsq_ref[...]
    sel = (lax.broadcasted_iota(jnp.int32, (8, _RBF_PAD), 0) == 0)
    dsq = lax.dot_general(dsq8, sel.astype(jnp.float32),
                          (((0,), (0,)), ((), ())),
                          preferred_element_type=jnp.float32)
    d = jnp.sqrt(dsq + 1e-12)
    step = float(_OFFS[1] - _OFFS[0])
    offs = lax.broadcasted_iota(jnp.int32, (1, _RBF_PAD), 1).astype(
        jnp.float32) * step
    f = jnp.exp(_COEFF * (d - offs) ** 2)
    m1 = lax.dot_general(f, w1_ref[0], (((1,), (0,)), ((), ())),
                         preferred_element_type=jnp.float32) + b1_ref[0]
    u = _ssp(m1)
    wij = lax.dot_general(u, w2_ref[0], (((1,), (0,)), ((), ())),
                          preferred_element_type=jnp.float32) + b2_ref[0]
    rc = 0.5 * (jnp.cos(d * (np.pi / CUTOFF)) + 1.0)
    rc = rc * (d < CUTOFF).astype(jnp.float32)
    rc = rc * (dsq < (3.0 * CUTOFF) ** 2).astype(jnp.float32)
    rc128 = lax.dot_general(rc, jnp.full((_RBF_PAD, 128), 1.0 / _RBF_PAD,
                                         jnp.float32),
                            (((1,), (0,)), ((), ())),
                            preferred_element_type=jnp.float32)
    eid = b * _BE + lax.broadcasted_iota(jnp.int32, (_BE, 128), 0)
    mask = (eid < N_EDGES).astype(jnp.float32)
    out_ref[0] = wij * rc128 * mask


_filt_call = pl.pallas_call(
    _filt_body,
    grid=(N_INTER, _NB),
    in_specs=[
        pl.BlockSpec((8, _BE), lambda l, b: (0, b)),
        pl.BlockSpec((1, _RBF_PAD, N_FILTERS), lambda l, b: (l, 0, 0)),
        pl.BlockSpec((1, 1, N_FILTERS), lambda l, b: (l, 0, 0)),
        pl.BlockSpec((1, N_FILTERS, N_FILTERS), lambda l, b: (l, 0, 0)),
        pl.BlockSpec((1, 1, N_FILTERS), lambda l, b: (l, 0, 0)),
    ],
    out_specs=pl.BlockSpec((1, _BE, N_FILTERS), lambda l, b: (l, b, 0)),
    out_shape=jax.ShapeDtypeStruct((N_INTER, _EPAD, N_FILTERS), jnp.float32),
)


def _lin_body(x_ref, w_ref, b_ref, out_ref):
    out_ref[...] = lax.dot_general(
        x_ref[...], w_ref[...], (((1,), (0,)), ((), ())),
        preferred_element_type=jnp.float32) + b_ref[0]


_h0_call = pl.pallas_call(
    _lin_body,
    grid=(_NP_A // _BE,),
    in_specs=[
        pl.BlockSpec((_BE, N_BASIS), lambda b: (b, 0)),
        pl.BlockSpec((N_BASIS, N_FILTERS), lambda b: (0, 0)),
        pl.BlockSpec((1, N_FILTERS), lambda b: (0, 0)),
    ],
    out_specs=pl.BlockSpec((_BE, N_FILTERS), lambda b: (b, 0)),
    out_shape=jax.ShapeDtypeStruct((_NP_A, N_FILTERS), jnp.float32),
)


def _out_body(pa_ref, pb_ref, x_ref, wo1_ref, bo1_ref, wo2_ref, bo2_ref,
              wn_ref, bn_ref, xn_ref, hn_ref):
    s = pa_ref[0] + pb_ref[0]
    u = _ssp(lax.dot_general(s, wo1_ref[...], (((1,), (0,)), ((), ())),
                             preferred_element_type=jnp.float32) + bo1_ref[0])
    v = lax.dot_general(u, wo2_ref[...], (((1,), (0,)), ((), ())),
                        preferred_element_type=jnp.float32) + bo2_ref[0]
    xn = x_ref[...] + v
    xn_ref[...] = xn
    hn_ref[...] = lax.dot_general(xn, wn_ref[...], (((1,), (0,)), ((), ())),
                                  preferred_element_type=jnp.float32) + bn_ref[0]


def _part_spec():
    def imap(b):
        p = b // 3
        return (p, b - 3 * p, 0)
    return pl.BlockSpec((1, _BR, N_FILTERS), imap)


_out_call = pl.pallas_call(
    _out_body,
    grid=(N_ATOMS // _BR,),
    in_specs=[
        _part_spec(),
        _part_spec(),
        pl.BlockSpec((_BR, N_BASIS), lambda b: (b, 0)),
        pl.BlockSpec((N_FILTERS, N_BASIS), lambda b: (0, 0)),
        pl.BlockSpec((1, N_BASIS), lambda b: (0, 0)),
        pl.BlockSpec((N_BASIS, N_BASIS), lambda b: (0, 0)),
        pl.BlockSpec((1, N_BASIS), lambda b: (0, 0)),
        pl.BlockSpec((N_BASIS, N_FILTERS), lambda b: (0, 0)),
        pl.BlockSpec((1, N_FILTERS), lambda b: (0, 0)),
    ],
    out_specs=[
        pl.BlockSpec((_BR, N_BASIS), lambda b: (b, 0)),
        pl.BlockSpec((_BR, N_FILTERS), lambda b: (b, 0)),
    ],
    out_shape=[
        jax.ShapeDtypeStruct((N_ATOMS, N_BASIS), jnp.float32),
        jax.ShapeDtypeStruct((_NP_A, N_FILTERS), jnp.float32),
    ],
)


def kernel(Z, R, atom_index12, emb, Win_w, Win_b, Wf1_w, Wf1_b, Wf2_w, Wf2_b,
           Wo1_w, Wo1_b, Wo2_w, Wo2_b):
    npad = _EPAD - N_EDGES
    idx_i = atom_index12[0].astype(jnp.int32)
    idx_j = atom_index12[1].astype(jnp.int32)
    idx_i = jnp.concatenate([idx_i, jnp.zeros((npad,), jnp.int32)])
    idx_j = jnp.concatenate([idx_j, jnp.zeros((npad,), jnp.int32)])
    idxi3d = idx_i.reshape(_NW, _NCHUNK, _CH)

    Rx = R[:, 0] + 0.0
    Ry = R[:, 1] + 0.0
    Rz = R[:, 2] + 0.0
    Zp = jnp.concatenate([Z.astype(jnp.int32),
                          jnp.zeros((_NP_A - N_ATOMS,), jnp.int32)])

    dsq8, x0p = _prep_call(Rx, Ry, Rz, idx_i, idx_j, Zp, emb)

    w1p = jnp.pad(Wf1_w, ((0, 0), (0, _RBF_PAD - N_RBF), (0, 0)))
    wij_all = _filt_call(dsq8, w1p, Wf1_b[:, None], Wf2_w, Wf2_b[:, None])

    h = _h0_call(x0p, Win_w[0], Win_b[0].reshape(1, -1))
    x = x0p[:N_ATOMS]
    for l in range(N_INTER):
        parts = _edge_call(h, wij_all[l].reshape(-1, _CH, N_FILTERS),
                           idxi3d, idx_j)
        nl = (l + 1) % N_INTER
        x, h = _out_call(parts[0], parts[1], x,
                         Wo1_w[l], Wo1_b[l].reshape(1, -1),
                         Wo2_w[l], Wo2_b[l].reshape(1, -1),
                         Win_w[nl], Win_b[nl].reshape(1, -1))
    return x


# speedup vs baseline: 1.6856x; 1.6856x over previous
import functools

import jax
import jax.numpy as jnp
import numpy as np
from jax import lax
from jax.experimental import pallas as pl
from jax.experimental.pallas import tpu as pltpu
from jax.experimental.pallas import tpu_sc as plsc

N_ATOMS = 10000
N_EDGES = 320000
N_BASIS = 128
N_FILTERS = 128
N_INTER = 3
N_RBF = 20
CUTOFF = 5.0

_NC = 2
_NS = 16
_NW = _NC * _NS
_CH = 128
_NCHUNK = 79
_EPW = _CH * _NCHUNK
_EPAD = _EPW * _NW
_HALF0 = 6000
_TRASH0 = 6000
_TRASH1 = 4112
_ACC = 6400
_RPT = _ACC // _NS

_NP_A = 10240

_RBF_PAD = 24
_OFFS = np.linspace(0.0, CUTOFF, N_RBF)
_COEFF = -0.5 / (_OFFS[1] - _OFFS[0]) ** 2
_OFFS_P = np.zeros((1, _RBF_PAD), np.float32)
_OFFS_P[0, :N_RBF] = _OFFS
_LOG2 = float(np.log(2.0))

_BE = 2048
_NB = _EPAD // _BE
_BR = 2000


def _ssp(x):
    return jnp.maximum(x, 0.0) + jnp.log1p(jnp.exp(-jnp.abs(x))) - _LOG2


_PCH = 1264
_ZPT = _NP_A // _NW


def _prep_body(rx_hbm, ry_hbm, rz_hbm, idxi_hbm, idxj_hbm, z_hbm, emb_hbm,
               dsq_hbm, x0_hbm,
               rx_v, ry_v, rz_v, ii_v, jj_v, dsq_v, zz_v, x0_v, sem):
    cid = lax.axis_index("c")
    sid = lax.axis_index("s")
    wid = cid * _NS + sid
    e0 = wid * _EPW

    pltpu.sync_copy(rx_hbm, rx_v)
    pltpu.sync_copy(ry_hbm, ry_v)
    pltpu.sync_copy(rz_hbm, rz_v)

    pltpu.sync_copy(z_hbm.at[pl.ds(wid * _ZPT, _ZPT)], zz_v)
    for q in range(4):
        pltpu.async_copy(emb_hbm.at[zz_v.at[pl.ds(q * 80, 80)]], x0_v,
                         sem).wait()
        pltpu.sync_copy(x0_v, x0_hbm.at[pl.ds(wid * _ZPT + q * 80, 80)])

    zf = jnp.zeros((16,), jnp.float32)

    def zrow(i, carry):
        for r in range(1, 8):
            dsq_v[r, pl.ds(i * 16, 16)] = zf
        return carry

    lax.fori_loop(0, _EPW // 16, zrow, 0)

    for c in range(8):
        pltpu.sync_copy(idxi_hbm.at[pl.ds(e0 + c * _PCH, _PCH)], ii_v)
        pltpu.sync_copy(idxj_hbm.at[pl.ds(e0 + c * _PCH, _PCH)], jj_v)

        def grp(g, carry):
            sl = pl.ds(g * 16, 16)
            vi = ii_v[sl]
            vj = jj_v[sl]
            dx = plsc.load_gather(rx_v, [vi]) - plsc.load_gather(rx_v, [vj])
            dy = plsc.load_gather(ry_v, [vi]) - plsc.load_gather(ry_v, [vj])
            dz = plsc.load_gather(rz_v, [vi]) - plsc.load_gather(rz_v, [vj])
            dsq_v[0, pl.ds(c * _PCH + g * 16, 16)] = dx * dx + dy * dy + dz * dz
            return carry

        lax.fori_loop(0, _PCH // 16, grp, 0)

    pltpu.sync_copy(dsq_v, dsq_hbm.at[:, pl.ds(e0, _EPW)])


_prep_call = functools.partial(
    pl.kernel,
    out_type=(jax.ShapeDtypeStruct((8, _EPAD), jnp.float32),
              jax.ShapeDtypeStruct((_NP_A, N_BASIS), jnp.float32)),
    mesh=plsc.VectorSubcoreMesh(core_axis_name="c", subcore_axis_name="s"),
    compiler_params=pltpu.CompilerParams(needs_layout_passes=False),
    scratch_types=[
        pltpu.VMEM((N_ATOMS,), jnp.float32),
        pltpu.VMEM((N_ATOMS,), jnp.float32),
        pltpu.VMEM((N_ATOMS,), jnp.float32),
        pltpu.VMEM((_PCH,), jnp.int32),
        pltpu.VMEM((_PCH,), jnp.int32),
        pltpu.VMEM((8, _EPW), jnp.float32),
        pltpu.VMEM((_ZPT,), jnp.int32),
        pltpu.VMEM((80, N_BASIS), jnp.float32),
        pltpu.SemaphoreType.DMA,
    ],
)(_prep_body)


def _edge_body(h_hbm, wij_hbm, idxi_hbm, idxj_hbm, out_hbm,
               idxa_v, idxb_v, idxj_v, xj_v, wij_v, s_sh, sem_g, sem_w):
    cid = lax.axis_index("c")
    sid = lax.axis_index("s")
    wid = cid * _NS + sid
    r0 = sid * _RPT

    pltpu.sync_copy(idxi_hbm.at[wid], idxa_v)
    pltpu.sync_copy(idxj_hbm.at[pl.ds(wid * _EPW, _EPW)], idxj_v)
    half = jnp.full((16,), _HALF0, jnp.int32)
    trash0 = jnp.full((16,), _TRASH0, jnp.int32)
    trash1 = jnp.full((16,), _TRASH1, jnp.int32)

    def remap_row(j, carry):
        for k in range(8):
            sl = pl.ds(k * 16, 16)
            v = idxa_v[j, sl]
            in_a = v < half
            idxb_v[j, sl] = jnp.where(in_a, trash1, v - half)
            idxa_v[j, sl] = jnp.where(in_a, v, trash0)
        return carry

    lax.fori_loop(0, _NCHUNK, remap_row, 0)

    zf = jnp.zeros((16,), jnp.float32)

    def zero_row(r, carry):
        for k in range(8):
            xj_v[r, pl.ds(k * 16, 16)] = zf
        return carry

    for p in range(2):
        lax.fori_loop(0, _CH, zero_row, 0)
        for j in range(3):
            pltpu.sync_copy(xj_v, s_sh.at[pl.ds(r0 + j * _CH, _CH)])
        pltpu.sync_copy(xj_v.at[pl.ds(0, 16)],
                        s_sh.at[pl.ds(r0 + 384, 16)])
        plsc.subcore_barrier()

        idx_ref = idxa_v if p == 0 else idxb_v

        def do_chunk(j, carry):
            pltpu.async_copy(
                h_hbm.at[idxj_v.at[pl.ds(j * _CH, _CH)]],
                xj_v, sem_g)
            pltpu.async_copy(wij_hbm.at[wid * _NCHUNK + j],
                             wij_v, sem_w)
            pltpu.make_async_copy(
                h_hbm.at[idxj_v.at[pl.ds(j * _CH, _CH)]],
                xj_v, sem_g).wait()
            pltpu.make_async_copy(wij_hbm.at[wid * _NCHUNK + j],
                                  wij_v, sem_w).wait()

            def mul_row(r, c2):
                for k in range(8):
                    sl = pl.ds(k * 16, 16)
                    xj_v[r, sl] = xj_v[r, sl] * wij_v[r, sl]
                return c2

            lax.fori_loop(0, _CH, mul_row, 0)
            pltpu.sync_copy(xj_v, s_sh.at[idx_ref.at[j]], add=True)
            return carry

        lax.fori_loop(0, _NCHUNK, do_chunk, 0)

        plsc.subcore_barrier()
        pltpu.sync_copy(s_sh.at[pl.ds(r0, _RPT)],
                        out_hbm.at[cid, p, pl.ds(r0, _RPT)])
        plsc.subcore_barrier()


_edge_call = functools.partial(
    pl.kernel,
    out_type=jax.ShapeDtypeStruct((_NC, 2, _ACC, N_FILTERS), jnp.float32),
    mesh=plsc.VectorSubcoreMesh(core_axis_name="c", subcore_axis_name="s"),
    scratch_types=[
        pltpu.VMEM((_NCHUNK, _CH), jnp.int32),
        pltpu.VMEM((_NCHUNK, _CH), jnp.int32),
        pltpu.VMEM((_EPW,), jnp.int32),
        pltpu.VMEM((_CH, N_FILTERS), jnp.float32),
        pltpu.VMEM((_CH, N_FILTERS), jnp.float32),
        pltpu.VMEM_SHARED((_ACC, N_FILTERS), jnp.float32),
        pltpu.SemaphoreType.DMA,
        pltpu.SemaphoreType.DMA,
    ],
)(_edge_body)


def _filt_body(dsq_ref, w1_ref, b1_ref, w2_ref, b2_ref, out_ref):
    b = pl.program_id(0)
    dsq8 = dsq_ref[...]
    sel = (lax.broadcasted_iota(jnp.int32, (8, _RBF_PAD), 0) == 0)
    dsq = lax.dot_general(dsq8, sel.astype(jnp.float32),
                          (((0,), (0,)), ((), ())),
                          preferred_element_type=jnp.float32)
    d = jnp.sqrt(dsq + 1e-12)
    step = float(_OFFS[1] - _OFFS[0])
    offs = lax.broadcasted_iota(jnp.int32, (1, _RBF_PAD), 1).astype(
        jnp.float32) * step
    f = jnp.exp(_COEFF * (d - offs) ** 2)
    m1 = lax.dot_general(f, w1_ref[0], (((1,), (0,)), ((), ())),
                         preferred_element_type=jnp.float32) + b1_ref[0]
    u = _ssp(m1)
    wij = lax.dot_general(u, w2_ref[0], (((1,), (0,)), ((), ())),
                          preferred_element_type=jnp.float32) + b2_ref[0]
    rc = 0.5 * (jnp.cos(d * (np.pi / CUTOFF)) + 1.0)
    rc = rc * (d < CUTOFF).astype(jnp.float32)
    rc = rc * (dsq < (3.0 * CUTOFF) ** 2).astype(jnp.float32)
    rc128 = lax.dot_general(rc, jnp.full((_RBF_PAD, 128), 1.0 / _RBF_PAD,
                                         jnp.float32),
                            (((1,), (0,)), ((), ())),
                            preferred_element_type=jnp.float32)
    eid = b * _BE + lax.broadcasted_iota(jnp.int32, (_BE, 128), 0)
    mask = (eid < N_EDGES).astype(jnp.float32)
    out_ref[...] = wij * rc128 * mask


_filt_call = pl.pallas_call(
    _filt_body,
    grid=(_NB,),
    in_specs=[
        pl.BlockSpec((8, _BE), lambda b: (0, b)),
        pl.BlockSpec((1, _RBF_PAD, N_FILTERS), lambda b: (0, 0, 0)),
        pl.BlockSpec((1, 1, N_FILTERS), lambda b: (0, 0, 0)),
        pl.BlockSpec((1, N_FILTERS, N_FILTERS), lambda b: (0, 0, 0)),
        pl.BlockSpec((1, 1, N_FILTERS), lambda b: (0, 0, 0)),
    ],
    out_specs=pl.BlockSpec((_BE, N_FILTERS), lambda b: (b, 0)),
    out_shape=jax.ShapeDtypeStruct((_EPAD, N_FILTERS), jnp.float32),
)


def _lin_body(x_ref, w_ref, b_ref, out_ref):
    out_ref[...] = lax.dot_general(
        x_ref[...], w_ref[...], (((1,), (0,)), ((), ())),
        preferred_element_type=jnp.float32) + b_ref[0]


_h0_call = pl.pallas_call(
    _lin_body,
    grid=(_NP_A // _BE,),
    in_specs=[
        pl.BlockSpec((_BE, N_BASIS), lambda b: (b, 0)),
        pl.BlockSpec((N_BASIS, N_FILTERS), lambda b: (0, 0)),
        pl.BlockSpec((1, N_FILTERS), lambda b: (0, 0)),
    ],
    out_specs=pl.BlockSpec((_BE, N_FILTERS), lambda b: (b, 0)),
    out_shape=jax.ShapeDtypeStruct((_NP_A, N_FILTERS), jnp.float32),
)


def _out_body(pa_ref, pb_ref, x_ref, wo1_ref, bo1_ref, wo2_ref, bo2_ref,
              wn_ref, bn_ref, xn_ref, hn_ref):
    s = pa_ref[0] + pb_ref[0]
    u = _ssp(lax.dot_general(s, wo1_ref[...], (((1,), (0,)), ((), ())),
                             preferred_element_type=jnp.float32) + bo1_ref[0])
    v = lax.dot_general(u, wo2_ref[...], (((1,), (0,)), ((), ())),
                        preferred_element_type=jnp.float32) + bo2_ref[0]
    xn = x_ref[...] + v
    xn_ref[...] = xn
    hn_ref[...] = lax.dot_general(xn, wn_ref[...], (((1,), (0,)), ((), ())),
                                  preferred_element_type=jnp.float32) + bn_ref[0]


def _part_spec():
    def imap(b):
        p = b // 3
        return (p, b - 3 * p, 0)
    return pl.BlockSpec((1, _BR, N_FILTERS), imap)


_out_call = pl.pallas_call(
    _out_body,
    grid=(N_ATOMS // _BR,),
    in_specs=[
        _part_spec(),
        _part_spec(),
        pl.BlockSpec((_BR, N_BASIS), lambda b: (b, 0)),
        pl.BlockSpec((N_FILTERS, N_BASIS), lambda b: (0, 0)),
        pl.BlockSpec((1, N_BASIS), lambda b: (0, 0)),
        pl.BlockSpec((N_BASIS, N_BASIS), lambda b: (0, 0)),
        pl.BlockSpec((1, N_BASIS), lambda b: (0, 0)),
        pl.BlockSpec((N_BASIS, N_FILTERS), lambda b: (0, 0)),
        pl.BlockSpec((1, N_FILTERS), lambda b: (0, 0)),
    ],
    out_specs=[
        pl.BlockSpec((_BR, N_BASIS), lambda b: (b, 0)),
        pl.BlockSpec((_BR, N_FILTERS), lambda b: (b, 0)),
    ],
    out_shape=[
        jax.ShapeDtypeStruct((N_ATOMS, N_BASIS), jnp.float32),
        jax.ShapeDtypeStruct((_NP_A, N_FILTERS), jnp.float32),
    ],
)


def kernel(Z, R, atom_index12, emb, Win_w, Win_b, Wf1_w, Wf1_b, Wf2_w, Wf2_b,
           Wo1_w, Wo1_b, Wo2_w, Wo2_b):
    npad = _EPAD - N_EDGES
    idx_i = atom_index12[0].astype(jnp.int32)
    idx_j = atom_index12[1].astype(jnp.int32)
    idx_i = jnp.concatenate([idx_i, jnp.zeros((npad,), jnp.int32)])
    idx_j = jnp.concatenate([idx_j, jnp.zeros((npad,), jnp.int32)])
    idxi3d = idx_i.reshape(_NW, _NCHUNK, _CH)

    Rx = R[:, 0] + 0.0
    Ry = R[:, 1] + 0.0
    Rz = R[:, 2] + 0.0
    Zp = jnp.concatenate([Z.astype(jnp.int32),
                          jnp.zeros((_NP_A - N_ATOMS,), jnp.int32)])

    dsq8, x0p = _prep_call(Rx, Ry, Rz, idx_i, idx_j, Zp, emb)

    w1p = jnp.pad(Wf1_w, ((0, 0), (0, _RBF_PAD - N_RBF), (0, 0)))
    wij_l = [_filt_call(dsq8, w1p[l:l + 1], Wf1_b[l:l + 1, None],
                        Wf2_w[l:l + 1], Wf2_b[l:l + 1, None])
             for l in range(N_INTER)]

    h = _h0_call(x0p, Win_w[0], Win_b[0].reshape(1, -1))
    x = x0p[:N_ATOMS]
    for l in range(N_INTER):
        parts = _edge_call(h, wij_l[l].reshape(-1, _CH, N_FILTERS),
                           idxi3d, idx_j)
        nl = (l + 1) % N_INTER
        x, h = _out_call(parts[0], parts[1], x,
                         Wo1_w[l], Wo1_b[l].reshape(1, -1),
                         Wo2_w[l], Wo2_b[l].reshape(1, -1),
                         Win_w[nl], Win_b[nl].reshape(1, -1))
    return x


# speedup vs baseline: 1.8085x; 1.0729x over previous
import functools

import jax
import jax.numpy as jnp
import numpy as np
from jax import lax
from jax.experimental import pallas as pl
from jax.experimental.pallas import tpu as pltpu
from jax.experimental.pallas import tpu_sc as plsc

N_ATOMS = 10000
N_EDGES = 320000
N_BASIS = 128
N_FILTERS = 128
N_INTER = 3
N_RBF = 20
CUTOFF = 5.0

_NC = 2
_NS = 16
_NW = _NC * _NS
_CH = 128
_NCHUNK = 79
_EPW = _CH * _NCHUNK
_EPAD = _EPW * _NW
_HALF0 = 6000
_TRASH0 = 6000
_TRASH1 = 4112
_ACC = 6400
_RPT = _ACC // _NS

_NP_A = 10240

_RBF_PAD = 24
_OFFS = np.linspace(0.0, CUTOFF, N_RBF)
_COEFF = -0.5 / (_OFFS[1] - _OFFS[0]) ** 2
_OFFS_P = np.zeros((1, _RBF_PAD), np.float32)
_OFFS_P[0, :N_RBF] = _OFFS
_LOG2 = float(np.log(2.0))

_BE = 2048
_NB = _EPAD // _BE
_BR = 2000


def _ssp(x):
    return jnp.maximum(x, 0.0) + jnp.log1p(jnp.exp(-jnp.abs(x))) - _LOG2


_PCH = 1264
_ZPT = _NP_A // _NW


def _prep_body(rx_hbm, ry_hbm, rz_hbm, idxi_hbm, idxj_hbm, z_hbm, emb_hbm,
               dsq_hbm, x0_hbm,
               rx_v, ry_v, rz_v, ii_v, jj_v, dsq_v, zz_v, x0_v, sem):
    cid = lax.axis_index("c")
    sid = lax.axis_index("s")
    wid = cid * _NS + sid
    e0 = wid * _EPW

    pltpu.sync_copy(rx_hbm, rx_v)
    pltpu.sync_copy(ry_hbm, ry_v)
    pltpu.sync_copy(rz_hbm, rz_v)

    pltpu.sync_copy(z_hbm.at[pl.ds(wid * _ZPT, _ZPT)], zz_v)
    for q in range(4):
        pltpu.async_copy(emb_hbm.at[zz_v.at[pl.ds(q * 80, 80)]], x0_v,
                         sem).wait()
        pltpu.sync_copy(x0_v, x0_hbm.at[pl.ds(wid * _ZPT + q * 80, 80)])

    zf = jnp.zeros((16,), jnp.float32)

    def zrow(i, carry):
        for r in range(1, 8):
            dsq_v[r, pl.ds(i * 16, 16)] = zf
        return carry

    lax.fori_loop(0, _EPW // 16, zrow, 0)

    for c in range(8):
        pltpu.sync_copy(idxi_hbm.at[pl.ds(e0 + c * _PCH, _PCH)], ii_v)
        pltpu.sync_copy(idxj_hbm.at[pl.ds(e0 + c * _PCH, _PCH)], jj_v)

        def grp(g, carry):
            sl = pl.ds(g * 16, 16)
            vi = ii_v[sl]
            vj = jj_v[sl]
            dx = plsc.load_gather(rx_v, [vi]) - plsc.load_gather(rx_v, [vj])
            dy = plsc.load_gather(ry_v, [vi]) - plsc.load_gather(ry_v, [vj])
            dz = plsc.load_gather(rz_v, [vi]) - plsc.load_gather(rz_v, [vj])
            dsq_v[0, pl.ds(c * _PCH + g * 16, 16)] = dx * dx + dy * dy + dz * dz
            return carry

        lax.fori_loop(0, _PCH // 16, grp, 0)

    pltpu.sync_copy(dsq_v, dsq_hbm.at[:, pl.ds(e0, _EPW)])


_prep_call = functools.partial(
    pl.kernel,
    out_type=(jax.ShapeDtypeStruct((8, _EPAD), jnp.float32),
              jax.ShapeDtypeStruct((_NP_A, N_BASIS), jnp.float32)),
    mesh=plsc.VectorSubcoreMesh(core_axis_name="c", subcore_axis_name="s"),
    compiler_params=pltpu.CompilerParams(needs_layout_passes=False),
    scratch_types=[
        pltpu.VMEM((N_ATOMS,), jnp.float32),
        pltpu.VMEM((N_ATOMS,), jnp.float32),
        pltpu.VMEM((N_ATOMS,), jnp.float32),
        pltpu.VMEM((_PCH,), jnp.int32),
        pltpu.VMEM((_PCH,), jnp.int32),
        pltpu.VMEM((8, _EPW), jnp.float32),
        pltpu.VMEM((_ZPT,), jnp.int32),
        pltpu.VMEM((80, N_BASIS), jnp.float32),
        pltpu.SemaphoreType.DMA,
    ],
)(_prep_body)


def _edge_body(h_hbm, wij_hbm, idxi_hbm, idxj_hbm, out_hbm,
               idxa_v, idxb_v, idxj_v, xj_v, wij_v, xj2_v, s_sh,
               sem_g, sem_w):
    cid = lax.axis_index("c")
    sid = lax.axis_index("s")
    wid = cid * _NS + sid
    r0 = sid * _RPT

    pltpu.sync_copy(idxi_hbm.at[wid], idxa_v)
    pltpu.sync_copy(idxj_hbm.at[pl.ds(wid * _EPW, _EPW)], idxj_v)
    half = jnp.full((16,), _HALF0, jnp.int32)
    trash0 = jnp.full((16,), _TRASH0, jnp.int32)
    trash1 = jnp.full((16,), _TRASH1, jnp.int32)

    def remap_row(j, carry):
        for k in range(8):
            sl = pl.ds(k * 16, 16)
            v = idxa_v[j, sl]
            in_a = v < half
            idxb_v[j, sl] = jnp.where(in_a, trash1, v - half)
            idxa_v[j, sl] = jnp.where(in_a, v, trash0)
        return carry

    lax.fori_loop(0, _NCHUNK, remap_row, 0)

    zf = jnp.zeros((16,), jnp.float32)

    def zero_row(r, carry):
        for k in range(8):
            xj_v[r, pl.ds(k * 16, 16)] = zf
        return carry

    for p in range(2):
        lax.fori_loop(0, _CH, zero_row, 0)
        for j in range(3):
            pltpu.sync_copy(xj_v, s_sh.at[pl.ds(r0 + j * _CH, _CH)])
        pltpu.sync_copy(xj_v.at[pl.ds(0, 16)],
                        s_sh.at[pl.ds(r0 + 384, 16)])
        plsc.subcore_barrier()

        idx_ref = idxa_v if p == 0 else idxb_v

        def issue(j, xjb):
            pltpu.async_copy(
                h_hbm.at[idxj_v.at[pl.ds(j * _CH, _CH)]], xjb, sem_g)

        def do_chunk(j, xjc, xjn):
            issue(jnp.minimum(j + 1, _NCHUNK - 1), xjn)
            pltpu.async_copy(wij_hbm.at[wid * _NCHUNK + j], wij_v, sem_w)
            pltpu.make_async_copy(
                h_hbm.at[idxj_v.at[pl.ds(j * _CH, _CH)]], xjc, sem_g).wait()
            pltpu.make_async_copy(wij_hbm.at[wid * _NCHUNK + j], wij_v,
                                  sem_w).wait()

            def mul_row(r, c2):
                for k in range(8):
                    sl = pl.ds(k * 16, 16)
                    xjc[r, sl] = xjc[r, sl] * wij_v[r, sl]
                return c2

            lax.fori_loop(0, _CH, mul_row, 0)
            pltpu.sync_copy(xjc, s_sh.at[idx_ref.at[j]], add=True)

        issue(0, xj_v)

        def pair(i, carry):
            do_chunk(2 * i, xj_v, xj2_v)
            do_chunk(2 * i + 1, xj2_v, xj_v)
            return carry

        lax.fori_loop(0, _NCHUNK // 2, pair, 0)
        do_chunk(_NCHUNK - 1, xj_v, xj2_v)
        pltpu.make_async_copy(
            h_hbm.at[idxj_v.at[pl.ds((_NCHUNK - 1) * _CH, _CH)]], xj2_v,
            sem_g).wait()

        plsc.subcore_barrier()
        pltpu.sync_copy(s_sh.at[pl.ds(r0, _RPT)],
                        out_hbm.at[cid, p, pl.ds(r0, _RPT)])
        plsc.subcore_barrier()


_edge_call = functools.partial(
    pl.kernel,
    out_type=jax.ShapeDtypeStruct((_NC, 2, _ACC, N_FILTERS), jnp.float32),
    mesh=plsc.VectorSubcoreMesh(core_axis_name="c", subcore_axis_name="s"),
    scratch_types=[
        pltpu.VMEM((_NCHUNK, _CH), jnp.int32),
        pltpu.VMEM((_NCHUNK, _CH), jnp.int32),
        pltpu.VMEM((_EPW,), jnp.int32),
        pltpu.VMEM((_CH, N_FILTERS), jnp.float32),
        pltpu.VMEM((_CH, N_FILTERS), jnp.float32),
        pltpu.VMEM((_CH, N_FILTERS), jnp.float32),
        pltpu.VMEM_SHARED((_ACC, N_FILTERS), jnp.float32),
        pltpu.SemaphoreType.DMA,
        pltpu.SemaphoreType.DMA,
    ],
)(_edge_body)


def _filt_body(dsq_ref, w1_ref, b1_ref, w2_ref, b2_ref, out_ref):
    b = pl.program_id(0)
    dsq8 = dsq_ref[...]
    sel = (lax.broadcasted_iota(jnp.int32, (8, _RBF_PAD), 0) == 0)
    dsq = lax.dot_general(dsq8, sel.astype(jnp.float32),
                          (((0,), (0,)), ((), ())),
                          preferred_element_type=jnp.float32)
    d = jnp.sqrt(dsq + 1e-12)
    step = float(_OFFS[1] - _OFFS[0])
    offs = lax.broadcasted_iota(jnp.int32, (1, _RBF_PAD), 1).astype(
        jnp.float32) * step
    f = jnp.exp(_COEFF * (d - offs) ** 2)
    m1 = lax.dot_general(f, w1_ref[0], (((1,), (0,)), ((), ())),
                         preferred_element_type=jnp.float32) + b1_ref[0]
    u = _ssp(m1)
    wij = lax.dot_general(u, w2_ref[0], (((1,), (0,)), ((), ())),
                          preferred_element_type=jnp.float32) + b2_ref[0]
    rc = 0.5 * (jnp.cos(d * (np.pi / CUTOFF)) + 1.0)
    rc = rc * (d < CUTOFF).astype(jnp.float32)
    rc = rc * (dsq < (3.0 * CUTOFF) ** 2).astype(jnp.float32)
    rc128 = lax.dot_general(rc, jnp.full((_RBF_PAD, 128), 1.0 / _RBF_PAD,
                                         jnp.float32),
                            (((1,), (0,)), ((), ())),
                            preferred_element_type=jnp.float32)
    eid = b * _BE + lax.broadcasted_iota(jnp.int32, (_BE, 128), 0)
    mask = (eid < N_EDGES).astype(jnp.float32)
    out_ref[...] = wij * rc128 * mask


_filt_call = pl.pallas_call(
    _filt_body,
    grid=(_NB,),
    in_specs=[
        pl.BlockSpec((8, _BE), lambda b: (0, b)),
        pl.BlockSpec((1, _RBF_PAD, N_FILTERS), lambda b: (0, 0, 0)),
        pl.BlockSpec((1, 1, N_FILTERS), lambda b: (0, 0, 0)),
        pl.BlockSpec((1, N_FILTERS, N_FILTERS), lambda b: (0, 0, 0)),
        pl.BlockSpec((1, 1, N_FILTERS), lambda b: (0, 0, 0)),
    ],
    out_specs=pl.BlockSpec((_BE, N_FILTERS), lambda b: (b, 0)),
    out_shape=jax.ShapeDtypeStruct((_EPAD, N_FILTERS), jnp.float32),
)


def _lin_body(x_ref, w_ref, b_ref, out_ref):
    out_ref[...] = lax.dot_general(
        x_ref[...], w_ref[...], (((1,), (0,)), ((), ())),
        preferred_element_type=jnp.float32) + b_ref[0]


_h0_call = pl.pallas_call(
    _lin_body,
    grid=(_NP_A // _BE,),
    in_specs=[
        pl.BlockSpec((_BE, N_BASIS), lambda b: (b, 0)),
        pl.BlockSpec((N_BASIS, N_FILTERS), lambda b: (0, 0)),
        pl.BlockSpec((1, N_FILTERS), lambda b: (0, 0)),
    ],
    out_specs=pl.BlockSpec((_BE, N_FILTERS), lambda b: (b, 0)),
    out_shape=jax.ShapeDtypeStruct((_NP_A, N_FILTERS), jnp.float32),
)


def _out_body(pa_ref, pb_ref, x_ref, wo1_ref, bo1_ref, wo2_ref, bo2_ref,
              wn_ref, bn_ref, xn_ref, hn_ref):
    s = pa_ref[0] + pb_ref[0]
    u = _ssp(lax.dot_general(s, wo1_ref[...], (((1,), (0,)), ((), ())),
                             preferred_element_type=jnp.float32) + bo1_ref[0])
    v = lax.dot_general(u, wo2_ref[...], (((1,), (0,)), ((), ())),
                        preferred_element_type=jnp.float32) + bo2_ref[0]
    xn = x_ref[...] + v
    xn_ref[...] = xn
    hn_ref[...] = lax.dot_general(xn, wn_ref[...], (((1,), (0,)), ((), ())),
                                  preferred_element_type=jnp.float32) + bn_ref[0]


def _part_spec():
    def imap(b):
        p = b // 3
        return (p, b - 3 * p, 0)
    return pl.BlockSpec((1, _BR, N_FILTERS), imap)


_out_call = pl.pallas_call(
    _out_body,
    grid=(N_ATOMS // _BR,),
    in_specs=[
        _part_spec(),
        _part_spec(),
        pl.BlockSpec((_BR, N_BASIS), lambda b: (b, 0)),
        pl.BlockSpec((N_FILTERS, N_BASIS), lambda b: (0, 0)),
        pl.BlockSpec((1, N_BASIS), lambda b: (0, 0)),
        pl.BlockSpec((N_BASIS, N_BASIS), lambda b: (0, 0)),
        pl.BlockSpec((1, N_BASIS), lambda b: (0, 0)),
        pl.BlockSpec((N_BASIS, N_FILTERS), lambda b: (0, 0)),
        pl.BlockSpec((1, N_FILTERS), lambda b: (0, 0)),
    ],
    out_specs=[
        pl.BlockSpec((_BR, N_BASIS), lambda b: (b, 0)),
        pl.BlockSpec((_BR, N_FILTERS), lambda b: (b, 0)),
    ],
    out_shape=[
        jax.ShapeDtypeStruct((N_ATOMS, N_BASIS), jnp.float32),
        jax.ShapeDtypeStruct((_NP_A, N_FILTERS), jnp.float32),
    ],
)


def kernel(Z, R, atom_index12, emb, Win_w, Win_b, Wf1_w, Wf1_b, Wf2_w, Wf2_b,
           Wo1_w, Wo1_b, Wo2_w, Wo2_b):
    npad = _EPAD - N_EDGES
    idx_i = atom_index12[0].astype(jnp.int32)
    idx_j = atom_index12[1].astype(jnp.int32)
    idx_i = jnp.concatenate([idx_i, jnp.zeros((npad,), jnp.int32)])
    idx_j = jnp.concatenate([idx_j, jnp.zeros((npad,), jnp.int32)])
    idxi3d = idx_i.reshape(_NW, _NCHUNK, _CH)

    Rx = R[:, 0] + 0.0
    Ry = R[:, 1] + 0.0
    Rz = R[:, 2] + 0.0
    Zp = jnp.concatenate([Z.astype(jnp.int32),
                          jnp.zeros((_NP_A - N_ATOMS,), jnp.int32)])

    dsq8, x0p = _prep_call(Rx, Ry, Rz, idx_i, idx_j, Zp, emb)

    w1p = jnp.pad(Wf1_w, ((0, 0), (0, _RBF_PAD - N_RBF), (0, 0)))
    wij_l = [_filt_call(dsq8, w1p[l:l + 1], Wf1_b[l:l + 1, None],
                        Wf2_w[l:l + 1], Wf2_b[l:l + 1, None])
             for l in range(N_INTER)]

    h = _h0_call(x0p, Win_w[0], Win_b[0].reshape(1, -1))
    x = x0p[:N_ATOMS]
    for l in range(N_INTER):
        parts = _edge_call(h, wij_l[l].reshape(-1, _CH, N_FILTERS),
                           idxi3d, idx_j)
        nl = (l + 1) % N_INTER
        x, h = _out_call(parts[0], parts[1], x,
                         Wo1_w[l], Wo1_b[l].reshape(1, -1),
                         Wo2_w[l], Wo2_b[l].reshape(1, -1),
                         Win_w[nl], Win_b[nl].reshape(1, -1))
    return x


# speedup vs baseline: 1.8228x; 1.0079x over previous
import functools

import jax
import jax.numpy as jnp
import numpy as np
from jax import lax
from jax.experimental import pallas as pl
from jax.experimental.pallas import tpu as pltpu
from jax.experimental.pallas import tpu_sc as plsc

N_ATOMS = 10000
N_EDGES = 320000
N_BASIS = 128
N_FILTERS = 128
N_INTER = 3
N_RBF = 20
CUTOFF = 5.0

_NC = 2
_NS = 16
_NW = _NC * _NS
_CH = 128
_NCHUNK = 79
_EPW = _CH * _NCHUNK
_EPAD = _EPW * _NW
_HALF0 = 6000
_TRASH0 = 6000
_TRASH1 = 4112
_ACC = 6400
_RPT = _ACC // _NS

_NP_A = 10240

_RBF_PAD = 24
_OFFS = np.linspace(0.0, CUTOFF, N_RBF)
_COEFF = -0.5 / (_OFFS[1] - _OFFS[0]) ** 2
_OFFS_P = np.zeros((1, _RBF_PAD), np.float32)
_OFFS_P[0, :N_RBF] = _OFFS
_LOG2 = float(np.log(2.0))

_BE = 2048
_NB = _EPAD // _BE
_BR = 2000


def _ssp(x):
    return jnp.maximum(x, 0.0) + jnp.log1p(jnp.exp(-jnp.abs(x))) - _LOG2


_PCH = 1264
_ZPT = _NP_A // _NW


def _prep_body(rx_hbm, ry_hbm, rz_hbm, idxi_hbm, idxj_hbm, z_hbm, emb_hbm,
               dsq_hbm, x0_hbm,
               rx_v, ry_v, rz_v, ii_v, jj_v, dsq_v, zz_v, x0_v, sem):
    cid = lax.axis_index("c")
    sid = lax.axis_index("s")
    wid = cid * _NS + sid
    e0 = wid * _EPW

    pltpu.sync_copy(rx_hbm, rx_v)
    pltpu.sync_copy(ry_hbm, ry_v)
    pltpu.sync_copy(rz_hbm, rz_v)

    pltpu.sync_copy(z_hbm.at[pl.ds(wid * _ZPT, _ZPT)], zz_v)
    for q in range(4):
        pltpu.async_copy(emb_hbm.at[zz_v.at[pl.ds(q * 80, 80)]], x0_v,
                         sem).wait()
        pltpu.sync_copy(x0_v, x0_hbm.at[pl.ds(wid * _ZPT + q * 80, 80)])

    zf = jnp.zeros((16,), jnp.float32)

    def zrow(i, carry):
        for r in range(1, 8):
            dsq_v[r, pl.ds(i * 16, 16)] = zf
        return carry

    lax.fori_loop(0, _EPW // 16, zrow, 0)

    for c in range(8):
        pltpu.sync_copy(idxi_hbm.at[pl.ds(e0 + c * _PCH, _PCH)], ii_v)
        pltpu.sync_copy(idxj_hbm.at[pl.ds(e0 + c * _PCH, _PCH)], jj_v)

        def grp(g, carry):
            sl = pl.ds(g * 16, 16)
            vi = ii_v[sl]
            vj = jj_v[sl]
            dx = plsc.load_gather(rx_v, [vi]) - plsc.load_gather(rx_v, [vj])
            dy = plsc.load_gather(ry_v, [vi]) - plsc.load_gather(ry_v, [vj])
            dz = plsc.load_gather(rz_v, [vi]) - plsc.load_gather(rz_v, [vj])
            dsq_v[0, pl.ds(c * _PCH + g * 16, 16)] = dx * dx + dy * dy + dz * dz
            return carry

        lax.fori_loop(0, _PCH // 16, grp, 0)

    pltpu.sync_copy(dsq_v, dsq_hbm.at[:, pl.ds(e0, _EPW)])


_prep_call = functools.partial(
    pl.kernel,
    out_type=(jax.ShapeDtypeStruct((8, _EPAD), jnp.float32),
              jax.ShapeDtypeStruct((_NP_A, N_BASIS), jnp.float32)),
    mesh=plsc.VectorSubcoreMesh(core_axis_name="c", subcore_axis_name="s"),
    compiler_params=pltpu.CompilerParams(needs_layout_passes=False),
    scratch_types=[
        pltpu.VMEM((N_ATOMS,), jnp.float32),
        pltpu.VMEM((N_ATOMS,), jnp.float32),
        pltpu.VMEM((N_ATOMS,), jnp.float32),
        pltpu.VMEM((_PCH,), jnp.int32),
        pltpu.VMEM((_PCH,), jnp.int32),
        pltpu.VMEM((8, _EPW), jnp.float32),
        pltpu.VMEM((_ZPT,), jnp.int32),
        pltpu.VMEM((80, N_BASIS), jnp.float32),
        pltpu.SemaphoreType.DMA,
    ],
)(_prep_body)


def _edge_body(h_hbm, wij_hbm, idxi_hbm, idxj_hbm, out_hbm,
               idxa_v, idxb_v, idxj_v, xj_v, wij_v, xj2_v, s_sh,
               sem_g, sem_w):
    cid = lax.axis_index("c")
    sid = lax.axis_index("s")
    wid = cid * _NS + sid
    r0 = sid * _RPT

    pltpu.sync_copy(idxi_hbm.at[wid], idxa_v)
    pltpu.sync_copy(idxj_hbm.at[pl.ds(wid * _EPW, _EPW)], idxj_v)
    half = jnp.full((16,), _HALF0, jnp.int32)
    lane8 = lax.iota(jnp.int32, 16) * 8

    def remap_row(j, carry):
        for k in range(8):
            sl = pl.ds(k * 16, 16)
            trash0 = (_TRASH0 + 16) + lane8 + k
            trash1 = _TRASH1 + lane8 + k
            v = idxa_v[j, sl]
            in_a = v < half
            idxb_v[j, sl] = jnp.where(in_a, trash1, v - half)
            idxa_v[j, sl] = jnp.where(in_a, v, trash0)
        return carry

    lax.fori_loop(0, _NCHUNK, remap_row, 0)

    zf = jnp.zeros((16,), jnp.float32)

    def zero_row(r, carry):
        for k in range(8):
            xj_v[r, pl.ds(k * 16, 16)] = zf
        return carry

    for p in range(2):
        lax.fori_loop(0, _CH, zero_row, 0)
        for j in range(3):
            pltpu.sync_copy(xj_v, s_sh.at[pl.ds(r0 + j * _CH, _CH)])
        pltpu.sync_copy(xj_v.at[pl.ds(0, 16)],
                        s_sh.at[pl.ds(r0 + 384, 16)])
        plsc.subcore_barrier()

        idx_ref = idxa_v if p == 0 else idxb_v

        def issue(j, xjb):
            pltpu.async_copy(
                h_hbm.at[idxj_v.at[pl.ds(j * _CH, _CH)]], xjb, sem_g)

        def do_chunk(j, xjc, xjn):
            issue(jnp.minimum(j + 1, _NCHUNK - 1), xjn)
            pltpu.async_copy(wij_hbm.at[wid * _NCHUNK + j], wij_v, sem_w)
            pltpu.make_async_copy(
                h_hbm.at[idxj_v.at[pl.ds(j * _CH, _CH)]], xjc, sem_g).wait()
            pltpu.make_async_copy(wij_hbm.at[wid * _NCHUNK + j], wij_v,
                                  sem_w).wait()

            def mul_row(r, c2):
                for k in range(8):
                    sl = pl.ds(k * 16, 16)
                    xjc[r, sl] = xjc[r, sl] * wij_v[r, sl]
                return c2

            lax.fori_loop(0, _CH, mul_row, 0)
            pltpu.sync_copy(xjc, s_sh.at[idx_ref.at[j]], add=True)

        issue(0, xj_v)

        def pair(i, carry):
            do_chunk(2 * i, xj_v, xj2_v)
            do_chunk(2 * i + 1, xj2_v, xj_v)
            return carry

        lax.fori_loop(0, _NCHUNK // 2, pair, 0)
        do_chunk(_NCHUNK - 1, xj_v, xj2_v)
        pltpu.make_async_copy(
            h_hbm.at[idxj_v.at[pl.ds((_NCHUNK - 1) * _CH, _CH)]], xj2_v,
            sem_g).wait()

        plsc.subcore_barrier()
        pltpu.sync_copy(s_sh.at[pl.ds(r0, _RPT)],
                        out_hbm.at[cid, p, pl.ds(r0, _RPT)])
        plsc.subcore_barrier()


_edge_call = functools.partial(
    pl.kernel,
    out_type=jax.ShapeDtypeStruct((_NC, 2, _ACC, N_FILTERS), jnp.float32),
    mesh=plsc.VectorSubcoreMesh(core_axis_name="c", subcore_axis_name="s"),
    scratch_types=[
        pltpu.VMEM((_NCHUNK, _CH), jnp.int32),
        pltpu.VMEM((_NCHUNK, _CH), jnp.int32),
        pltpu.VMEM((_EPW,), jnp.int32),
        pltpu.VMEM((_CH, N_FILTERS), jnp.float32),
        pltpu.VMEM((_CH, N_FILTERS), jnp.float32),
        pltpu.VMEM((_CH, N_FILTERS), jnp.float32),
        pltpu.VMEM_SHARED((_ACC, N_FILTERS), jnp.float32),
        pltpu.SemaphoreType.DMA,
        pltpu.SemaphoreType.DMA,
    ],
)(_edge_body)


def _filt_body(dsq_ref, w1_ref, b1_ref, w2_ref, b2_ref, out_ref):
    b = pl.program_id(0)
    dsq8 = dsq_ref[...]
    sel = (lax.broadcasted_iota(jnp.int32, (8, _RBF_PAD), 0) == 0)
    dsq = lax.dot_general(dsq8, sel.astype(jnp.float32),
                          (((0,), (0,)), ((), ())),
                          preferred_element_type=jnp.float32)
    d = jnp.sqrt(dsq + 1e-12)
    step = float(_OFFS[1] - _OFFS[0])
    offs = lax.broadcasted_iota(jnp.int32, (1, _RBF_PAD), 1).astype(
        jnp.float32) * step
    f = jnp.exp(_COEFF * (d - offs) ** 2)
    m1 = lax.dot_general(f, w1_ref[0], (((1,), (0,)), ((), ())),
                         preferred_element_type=jnp.float32) + b1_ref[0]
    u = _ssp(m1)
    wij = lax.dot_general(u, w2_ref[0], (((1,), (0,)), ((), ())),
                          preferred_element_type=jnp.float32) + b2_ref[0]
    rc = 0.5 * (jnp.cos(d * (np.pi / CUTOFF)) + 1.0)
    rc = rc * (d < CUTOFF).astype(jnp.float32)
    rc = rc * (dsq < (3.0 * CUTOFF) ** 2).astype(jnp.float32)
    rc128 = lax.dot_general(rc, jnp.full((_RBF_PAD, 128), 1.0 / _RBF_PAD,
                                         jnp.float32),
                            (((1,), (0,)), ((), ())),
                            preferred_element_type=jnp.float32)
    eid = b * _BE + lax.broadcasted_iota(jnp.int32, (_BE, 128), 0)
    mask = (eid < N_EDGES).astype(jnp.float32)
    out_ref[...] = wij * rc128 * mask


_filt_call = pl.pallas_call(
    _filt_body,
    grid=(_NB,),
    in_specs=[
        pl.BlockSpec((8, _BE), lambda b: (0, b)),
        pl.BlockSpec((1, _RBF_PAD, N_FILTERS), lambda b: (0, 0, 0)),
        pl.BlockSpec((1, 1, N_FILTERS), lambda b: (0, 0, 0)),
        pl.BlockSpec((1, N_FILTERS, N_FILTERS), lambda b: (0, 0, 0)),
        pl.BlockSpec((1, 1, N_FILTERS), lambda b: (0, 0, 0)),
    ],
    out_specs=pl.BlockSpec((_BE, N_FILTERS), lambda b: (b, 0)),
    out_shape=jax.ShapeDtypeStruct((_EPAD, N_FILTERS), jnp.float32),
)


def _lin_body(x_ref, w_ref, b_ref, out_ref):
    out_ref[...] = lax.dot_general(
        x_ref[...], w_ref[...], (((1,), (0,)), ((), ())),
        preferred_element_type=jnp.float32) + b_ref[0]


_h0_call = pl.pallas_call(
    _lin_body,
    grid=(_NP_A // _BE,),
    in_specs=[
        pl.BlockSpec((_BE, N_BASIS), lambda b: (b, 0)),
        pl.BlockSpec((N_BASIS, N_FILTERS), lambda b: (0, 0)),
        pl.BlockSpec((1, N_FILTERS), lambda b: (0, 0)),
    ],
    out_specs=pl.BlockSpec((_BE, N_FILTERS), lambda b: (b, 0)),
    out_shape=jax.ShapeDtypeStruct((_NP_A, N_FILTERS), jnp.float32),
)


def _out_body(pa_ref, pb_ref, x_ref, wo1_ref, bo1_ref, wo2_ref, bo2_ref,
              wn_ref, bn_ref, xn_ref, hn_ref):
    s = pa_ref[0] + pb_ref[0]
    u = _ssp(lax.dot_general(s, wo1_ref[...], (((1,), (0,)), ((), ())),
                             preferred_element_type=jnp.float32) + bo1_ref[0])
    v = lax.dot_general(u, wo2_ref[...], (((1,), (0,)), ((), ())),
                        preferred_element_type=jnp.float32) + bo2_ref[0]
    xn = x_ref[...] + v
    xn_ref[...] = xn
    hn_ref[...] = lax.dot_general(xn, wn_ref[...], (((1,), (0,)), ((), ())),
                                  preferred_element_type=jnp.float32) + bn_ref[0]


def _part_spec():
    def imap(b):
        p = b // 3
        return (p, b - 3 * p, 0)
    return pl.BlockSpec((1, _BR, N_FILTERS), imap)


_out_call = pl.pallas_call(
    _out_body,
    grid=(N_ATOMS // _BR,),
    in_specs=[
        _part_spec(),
        _part_spec(),
        pl.BlockSpec((_BR, N_BASIS), lambda b: (b, 0)),
        pl.BlockSpec((N_FILTERS, N_BASIS), lambda b: (0, 0)),
        pl.BlockSpec((1, N_BASIS), lambda b: (0, 0)),
        pl.BlockSpec((N_BASIS, N_BASIS), lambda b: (0, 0)),
        pl.BlockSpec((1, N_BASIS), lambda b: (0, 0)),
        pl.BlockSpec((N_BASIS, N_FILTERS), lambda b: (0, 0)),
        pl.BlockSpec((1, N_FILTERS), lambda b: (0, 0)),
    ],
    out_specs=[
        pl.BlockSpec((_BR, N_BASIS), lambda b: (b, 0)),
        pl.BlockSpec((_BR, N_FILTERS), lambda b: (b, 0)),
    ],
    out_shape=[
        jax.ShapeDtypeStruct((N_ATOMS, N_BASIS), jnp.float32),
        jax.ShapeDtypeStruct((_NP_A, N_FILTERS), jnp.float32),
    ],
)


def kernel(Z, R, atom_index12, emb, Win_w, Win_b, Wf1_w, Wf1_b, Wf2_w, Wf2_b,
           Wo1_w, Wo1_b, Wo2_w, Wo2_b):
    npad = _EPAD - N_EDGES
    idx_i = atom_index12[0].astype(jnp.int32)
    idx_j = atom_index12[1].astype(jnp.int32)
    idx_i = jnp.concatenate([idx_i, jnp.zeros((npad,), jnp.int32)])
    idx_j = jnp.concatenate([idx_j, jnp.zeros((npad,), jnp.int32)])
    idxi3d = idx_i.reshape(_NW, _NCHUNK, _CH)

    Rx = R[:, 0] + 0.0
    Ry = R[:, 1] + 0.0
    Rz = R[:, 2] + 0.0
    Zp = jnp.concatenate([Z.astype(jnp.int32),
                          jnp.zeros((_NP_A - N_ATOMS,), jnp.int32)])

    dsq8, x0p = _prep_call(Rx, Ry, Rz, idx_i, idx_j, Zp, emb)

    w1p = jnp.pad(Wf1_w, ((0, 0), (0, _RBF_PAD - N_RBF), (0, 0)))
    wij_l = [_filt_call(dsq8, w1p[l:l + 1], Wf1_b[l:l + 1, None],
                        Wf2_w[l:l + 1], Wf2_b[l:l + 1, None])
             for l in range(N_INTER)]

    h = _h0_call(x0p, Win_w[0], Win_b[0].reshape(1, -1))
    x = x0p[:N_ATOMS]
    for l in range(N_INTER):
        parts = _edge_call(h, wij_l[l].reshape(-1, _CH, N_FILTERS),
                           idxi3d, idx_j)
        nl = (l + 1) % N_INTER
        x, h = _out_call(parts[0], parts[1], x,
                         Wo1_w[l], Wo1_b[l].reshape(1, -1),
                         Wo2_w[l], Wo2_b[l].reshape(1, -1),
                         Win_w[nl], Win_b[nl].reshape(1, -1))
    return x
